# baseline (speedup 1.0000x reference)
import jax
import jax.numpy as jnp
from jax.experimental import pallas as pl

_EPS = 1e-5
_MB = 256
_MB4 = 1024
_BIG = 3.0e38


def _acc(ref, val, first):
    @pl.when(first)
    def _():
        ref[...] = jnp.zeros_like(ref)

    ref[...] += val


def _dot(a, b):
    return jnp.dot(a, b, preferred_element_type=jnp.float32)


def _stats(hs, ms, s_ref, q_ref, first):
    hmacc = hs[0] * ms[0]
    qacc = hmacc * hs[0]
    for n in range(1, len(hs)):
        hm = hs[n] * ms[n]
        hmacc = hmacc + hm
        qacc = qacc + hm * hs[n]
    ones = jnp.ones((hmacc.shape[1], 1), jnp.float32)
    _acc(s_ref, _dot(hmacc, ones), first)
    _acc(q_ref, _dot(qacc, ones), first)


def _slabs(x_ref, m_ref, n_pts, c_in):
    xt = jnp.transpose(x_ref[...])
    mt = jnp.transpose(m_ref[...])
    xs = [xt[n * c_in:(n + 1) * c_in] for n in range(n_pts)]
    ms = [mt[n:n + 1] for n in range(n_pts)]
    return xs, ms


def _ph1(x_ref, m_ref, wp_ref, s_ref, q_ref, c_ref):
    first = pl.program_id(0) == 0
    wp = wp_ref[...]
    n_pts = m_ref.shape[1]
    xs, ms = _slabs(x_ref, m_ref, n_pts, wp.shape[1])
    hs = [_dot(wp, xs[n]) for n in range(n_pts)]
    _stats(hs, ms, s_ref, q_ref, first)
    _acc(c_ref, jnp.sum(m_ref[...]).reshape(1, 1), first)


def _ph2(x_ref, m_ref, wp_ref, sc0_ref, sh0_ref, w1a_ref, w1b_ref,
         s_ref, q_ref, h2_ref):
    first = pl.program_id(0) == 0
    wp = wp_ref[...]
    sc0 = sc0_ref[...]
    sh0 = sh0_ref[...]
    n_pts = m_ref.shape[1]
    xs, ms = _slabs(x_ref, m_ref, n_pts, wp.shape[1])
    feats = []
    for n in range(n_pts):
        h1 = _dot(wp, xs[n])
        feats.append(jnp.maximum(h1 * sc0 + sh0, 0.0) * ms[n])
    pooled = feats[0]
    for n in range(1, n_pts):
        pooled = jnp.maximum(pooled, feats[n])
    pb = _dot(w1b_ref[...], pooled)
    w1a = w1a_ref[...]
    h2s = []
    for n in range(n_pts):
        h2 = _dot(w1a, feats[n]) + pb
        h2_ref[n] = h2.astype(jnp.bfloat16)
        h2s.append(h2)
    _stats(h2s, ms, s_ref, q_ref, first)


def _ph3(h2_ref, m_ref, sc1_ref, sh1_ref, w2_ref, s_ref, q_ref, bm_ref):
    first = pl.program_id(0) == 0
    sc1 = sc1_ref[...]
    sh1 = sh1_ref[...]
    w2 = w2_ref[...]
    n_pts = h2_ref.shape[0]
    mt = jnp.transpose(m_ref[...])
    ms = [mt[n:n + 1] for n in range(n_pts)]
    h3s = []
    bmacc = None
    for n in range(n_pts):
        t2 = jnp.maximum(h2_ref[n].astype(jnp.float32) * sc1 + sh1, 0.0)
        h3 = _dot(w2, t2)
        h3s.append(h3)
        h3m = jnp.where(ms[n] != 0.0, h3, -_BIG)
        bmacc = h3m if bmacc is None else jnp.maximum(bmacc, h3m)
    bm_ref[...] = bmacc
    _stats(h3s, ms, s_ref, q_ref, first)


def _ph4(bm_ref, sc2_ref, sh2_ref, wo1_ref, bo1_ref, waug_ref, o_ref):
    bm = bm_ref[...]
    buf = jnp.maximum(bm * sc2_ref[...] + sh2_ref[...], 0.0)
    t = jnp.maximum(_dot(wo1_ref[...], buf) + bo1_ref[...], 0.0)
    validf = (bm[0:1, :] > -1e37).astype(jnp.float32)
    ta = jnp.concatenate([t * validf, validf], axis=0)
    o_ref[...] = _dot(ta.T, waug_ref[...])


def _scale_shift(s, q, cnt, g, b):
    mean = s[:, 0] / cnt
    var = jnp.maximum(q[:, 0] / cnt - mean * mean, 0.0)
    sc = g * jax.lax.rsqrt(var + _EPS)
    sh = b - mean * sc
    return sc[:, None], sh[:, None]


def kernel(polylines, polylines_mask, W_pre, g_pre, b_pre, W1, g1, b1,
           W2, g2, b2, Wo1, bo1, Wo2, bo2):
    B, P, N, C = polylines.shape
    H = W_pre.shape[0]
    O = Wo2.shape[0]
    M = B * P
    f32 = jnp.float32

    xf = polylines.reshape(M, N * C)
    mf = polylines_mask.reshape(M, N).astype(f32)
    waug = jnp.concatenate([Wo2.T, bo2[None, :]], axis=0)

    grid = (M // _MB,)
    x_spec = pl.BlockSpec((_MB, N * C), lambda i: (i, 0))
    m_spec = pl.BlockSpec((_MB, N), lambda i: (i, 0))

    def full(shp):
        return pl.BlockSpec(shp, lambda i: tuple(0 for _ in shp))

    accH = pl.BlockSpec((H, 1), lambda i: (0, 0))
    acc1 = pl.BlockSpec((1, 1), lambda i: (0, 0))
    sdH = jax.ShapeDtypeStruct((H, 1), f32)

    s0, q0, c0 = pl.pallas_call(
        _ph1,
        grid=grid,
        in_specs=[x_spec, m_spec, full((H, C))],
        out_specs=[accH, accH, acc1],
        out_shape=[sdH, sdH, jax.ShapeDtypeStruct((1, 1), f32)],
    )(xf, mf, W_pre)

    cnt = jnp.maximum(c0[0, 0], 1.0)
    sc0, sh0 = _scale_shift(s0, q0, cnt, g_pre, b_pre)

    h_spec = pl.BlockSpec((N, H, _MB), lambda i: (0, 0, i))

    s1, q1, h2buf = pl.pallas_call(
        _ph2,
        grid=grid,
        in_specs=[x_spec, m_spec, full((H, C)), full((H, 1)), full((H, 1)),
                  full((H, H)), full((H, H))],
        out_specs=[accH, accH, h_spec],
        out_shape=[sdH, sdH, jax.ShapeDtypeStruct((N, H, M), jnp.bfloat16)],
    )(xf, mf, W_pre, sc0, sh0, W1[:, :H], W1[:, H:])

    sc1, sh1 = _scale_shift(s1, q1, cnt, g1, b1)

    s2, q2, bmax = pl.pallas_call(
        _ph3,
        grid=grid,
        in_specs=[h_spec, m_spec, full((H, 1)), full((H, 1)), full((H, H))],
        out_specs=[accH, accH, pl.BlockSpec((H, _MB), lambda i: (0, i))],
        out_shape=[sdH, sdH, jax.ShapeDtypeStruct((H, M), f32)],
    )(h2buf, mf, sc1, sh1, W2)

    sc2, sh2 = _scale_shift(s2, q2, cnt, g2, b2)

    out = pl.pallas_call(
        _ph4,
        grid=(M // _MB4,),
        in_specs=[pl.BlockSpec((H, _MB4), lambda i: (0, i)),
                  full((H, 1)), full((H, 1)),
                  full((H, H)), full((H, 1)), full((H + 1, O))],
        out_specs=pl.BlockSpec((_MB4, O), lambda i: (i, 0)),
        out_shape=jax.ShapeDtypeStruct((M, O), f32),
    )(bmax, sc2, sh2, Wo1, bo1.reshape(H, 1), waug)

    return out.reshape(B, P, O)


# speedup vs baseline: 1.5546x; 1.5546x over previous
import jax
import jax.numpy as jnp
from jax.experimental import pallas as pl

_EPS = 1e-5
_MB = 512
_MB4 = 1024
_BIG = 3.0e38


def _acc(ref, val, first):
    @pl.when(first)
    def _():
        ref[...] = jnp.zeros_like(ref)

    ref[...] += val


def _dot(a, b):
    return jnp.dot(a, b, preferred_element_type=jnp.float32)


def _stats(hs, ms, s_ref, q_ref, first):
    hmacc = hs[0] * ms[0]
    qacc = hmacc * hs[0]
    for n in range(1, len(hs)):
        hm = hs[n] * ms[n]
        hmacc = hmacc + hm
        qacc = qacc + hm * hs[n]
    ones = jnp.ones((hmacc.shape[1], 1), jnp.float32)
    _acc(s_ref, _dot(hmacc, ones), first)
    _acc(q_ref, _dot(qacc, ones), first)


def _ph1(x_ref, m_ref, wp_ref, s_ref, q_ref, c_ref):
    first = pl.program_id(0) == 0
    wp = wp_ref[...]
    m = m_ref[...]
    hs = [_dot(wp, x_ref[n]) for n in range(x_ref.shape[0])]
    ms = [m[n] for n in range(m.shape[0])]
    _stats(hs, ms, s_ref, q_ref, first)
    _acc(c_ref, jnp.sum(m).reshape(1, 1), first)


def _ph2(x_ref, m_ref, wp_ref, sc0_ref, sh0_ref, w1a_ref, w1b_ref,
         s_ref, q_ref, h2_ref):
    first = pl.program_id(0) == 0
    wp = wp_ref[...]
    sc0 = sc0_ref[...]
    sh0 = sh0_ref[...]
    m = m_ref[...]
    n_pts = x_ref.shape[0]
    feats = []
    for n in range(n_pts):
        h1 = _dot(wp, x_ref[n])
        feats.append(jnp.maximum(h1 * sc0 + sh0, 0.0) * m[n])
    pooled = feats[0]
    for n in range(1, n_pts):
        pooled = jnp.maximum(pooled, feats[n])
    pb = _dot(w1b_ref[...], pooled)
    w1a = w1a_ref[...]
    h2s = []
    for n in range(n_pts):
        h2 = _dot(w1a, feats[n]) + pb
        h2_ref[n] = h2.astype(jnp.bfloat16)
        h2s.append(h2)
    _stats(h2s, [m[n] for n in range(n_pts)], s_ref, q_ref, first)


def _ph3(h2_ref, m_ref, sc1_ref, sh1_ref, w2_ref, s_ref, q_ref, bm_ref):
    first = pl.program_id(0) == 0
    sc1 = sc1_ref[...]
    sh1 = sh1_ref[...]
    w2 = w2_ref[...]
    m = m_ref[...]
    n_pts = h2_ref.shape[0]
    h3s = []
    bmacc = None
    for n in range(n_pts):
        t2 = jnp.maximum(h2_ref[n].astype(jnp.float32) * sc1 + sh1, 0.0)
        h3 = _dot(w2, t2)
        h3s.append(h3)
        h3m = jnp.where(m[n] != 0.0, h3, -_BIG)
        bmacc = h3m if bmacc is None else jnp.maximum(bmacc, h3m)
    bm_ref[...] = bmacc
    _stats(h3s, [m[n] for n in range(n_pts)], s_ref, q_ref, first)


def _ph4(bm_ref, sc2_ref, sh2_ref, wo1_ref, bo1_ref, waug_ref, o_ref):
    bm = bm_ref[...]
    buf = jnp.maximum(bm * sc2_ref[...] + sh2_ref[...], 0.0)
    t = jnp.maximum(_dot(wo1_ref[...], buf) + bo1_ref[...], 0.0)
    validf = (bm[0:1, :] > -1e37).astype(jnp.float32)
    ta = jnp.concatenate([t * validf, validf], axis=0)
    o_ref[...] = _dot(ta.T, waug_ref[...])


def _scale_shift(s, q, cnt, g, b):
    mean = s[:, 0] / cnt
    var = jnp.maximum(q[:, 0] / cnt - mean * mean, 0.0)
    sc = g * jax.lax.rsqrt(var + _EPS)
    sh = b - mean * sc
    return sc[:, None], sh[:, None]


def kernel(polylines, polylines_mask, W_pre, g_pre, b_pre, W1, g1, b1,
           W2, g2, b2, Wo1, bo1, Wo2, bo2):
    B, P, N, C = polylines.shape
    H = W_pre.shape[0]
    O = Wo2.shape[0]
    M = B * P
    f32 = jnp.float32

    xt = polylines.reshape(M, N, C).transpose(1, 2, 0)
    mt = polylines_mask.reshape(M, N).T.reshape(N, 1, M).astype(f32)
    waug = jnp.concatenate([Wo2.T, bo2[None, :]], axis=0)

    grid = (M // _MB,)
    x_spec = pl.BlockSpec((N, C, _MB), lambda i: (0, 0, i))
    m_spec = pl.BlockSpec((N, 1, _MB), lambda i: (0, 0, i))

    def full(shp):
        return pl.BlockSpec(shp, lambda i: tuple(0 for _ in shp))

    accH = pl.BlockSpec((H, 1), lambda i: (0, 0))
    acc1 = pl.BlockSpec((1, 1), lambda i: (0, 0))
    sdH = jax.ShapeDtypeStruct((H, 1), f32)

    s0, q0, c0 = pl.pallas_call(
        _ph1,
        grid=grid,
        in_specs=[x_spec, m_spec, full((H, C))],
        out_specs=[accH, accH, acc1],
        out_shape=[sdH, sdH, jax.ShapeDtypeStruct((1, 1), f32)],
    )(xt, mt, W_pre)

    cnt = jnp.maximum(c0[0, 0], 1.0)
    sc0, sh0 = _scale_shift(s0, q0, cnt, g_pre, b_pre)

    h_spec = pl.BlockSpec((N, H, _MB), lambda i: (0, 0, i))

    s1, q1, h2buf = pl.pallas_call(
        _ph2,
        grid=grid,
        in_specs=[x_spec, m_spec, full((H, C)), full((H, 1)), full((H, 1)),
                  full((H, H)), full((H, H))],
        out_specs=[accH, accH, h_spec],
        out_shape=[sdH, sdH, jax.ShapeDtypeStruct((N, H, M), jnp.bfloat16)],
    )(xt, mt, W_pre, sc0, sh0, W1[:, :H], W1[:, H:])

    sc1, sh1 = _scale_shift(s1, q1, cnt, g1, b1)

    s2, q2, bmax = pl.pallas_call(
        _ph3,
        grid=grid,
        in_specs=[h_spec, m_spec, full((H, 1)), full((H, 1)), full((H, H))],
        out_specs=[accH, accH, pl.BlockSpec((H, _MB), lambda i: (0, i))],
        out_shape=[sdH, sdH, jax.ShapeDtypeStruct((H, M), f32)],
    )(h2buf, mt, sc1, sh1, W2)

    sc2, sh2 = _scale_shift(s2, q2, cnt, g2, b2)

    out = pl.pallas_call(
        _ph4,
        grid=(M // _MB4,),
        in_specs=[pl.BlockSpec((H, _MB4), lambda i: (0, i)),
                  full((H, 1)), full((H, 1)),
                  full((H, H)), full((H, 1)), full((H + 1, O))],
        out_specs=pl.BlockSpec((_MB4, O), lambda i: (i, 0)),
        out_shape=jax.ShapeDtypeStruct((M, O), f32),
    )(bmax, sc2, sh2, Wo1, bo1.reshape(H, 1), waug)

    return out.reshape(B, P, O)


# speedup vs baseline: 1.6514x; 1.0623x over previous
import jax
import jax.numpy as jnp
from jax.experimental import pallas as pl

_EPS = 1e-5
_MB = 768
_MB4 = 1024
_BIG = 3.0e38


def _acc(ref, val, first):
    @pl.when(first)
    def _():
        ref[...] = jnp.zeros_like(ref)

    ref[...] += val


def _dot(a, b):
    return jnp.dot(a, b, preferred_element_type=jnp.float32)


def _stats(hs, ms, s_ref, q_ref, first):
    hmacc = hs[0] * ms[0]
    qacc = hmacc * hs[0]
    for n in range(1, len(hs)):
        hm = hs[n] * ms[n]
        hmacc = hmacc + hm
        qacc = qacc + hm * hs[n]
    ones = jnp.ones((hmacc.shape[1], 1), jnp.float32)
    _acc(s_ref, _dot(hmacc, ones), first)
    _acc(q_ref, _dot(qacc, ones), first)


def _ph1(x_ref, m_ref, wp_ref, s_ref, q_ref, c_ref):
    first = pl.program_id(0) == 0
    wp = wp_ref[...]
    m = m_ref[...]
    hs = [_dot(wp, x_ref[n]) for n in range(x_ref.shape[0])]
    ms = [m[n] for n in range(m.shape[0])]
    _stats(hs, ms, s_ref, q_ref, first)
    _acc(c_ref, jnp.sum(m).reshape(1, 1), first)


def _ph2(x_ref, m_ref, wp_ref, sc0_ref, sh0_ref, w1a_ref, w1b_ref,
         s_ref, q_ref, h2_ref):
    first = pl.program_id(0) == 0
    wp = wp_ref[...]
    sc0 = sc0_ref[...]
    sh0 = sh0_ref[...]
    m = m_ref[...]
    n_pts = x_ref.shape[0]
    feats = []
    for n in range(n_pts):
        h1 = _dot(wp, x_ref[n])
        feats.append(jnp.maximum(h1 * sc0 + sh0, 0.0) * m[n])
    pooled = feats[0]
    for n in range(1, n_pts):
        pooled = jnp.maximum(pooled, feats[n])
    pb = _dot(w1b_ref[...], pooled)
    w1a = w1a_ref[...]
    h2s = []
    for n in range(n_pts):
        h2 = _dot(w1a, feats[n]) + pb
        h2_ref[n] = h2.astype(jnp.bfloat16)
        h2s.append(h2)
    _stats(h2s, [m[n] for n in range(n_pts)], s_ref, q_ref, first)


def _ph3(h2_ref, m_ref, sc1_ref, sh1_ref, w2_ref, s_ref, q_ref, bm_ref):
    first = pl.program_id(0) == 0
    sc1 = sc1_ref[...]
    sh1 = sh1_ref[...]
    w2 = w2_ref[...]
    m = m_ref[...]
    n_pts = h2_ref.shape[0]
    h3s = []
    bmacc = None
    for n in range(n_pts):
        t2 = jnp.maximum(h2_ref[n].astype(jnp.float32) * sc1 + sh1, 0.0)
        h3 = _dot(w2, t2)
        h3s.append(h3)
        h3m = jnp.where(m[n] != 0.0, h3, -_BIG)
        bmacc = h3m if bmacc is None else jnp.maximum(bmacc, h3m)
    bm_ref[...] = bmacc
    _stats(h3s, [m[n] for n in range(n_pts)], s_ref, q_ref, first)


def _ph4(bm_ref, sc2_ref, sh2_ref, wo1_ref, bo1_ref, waug_ref, o_ref):
    bm = bm_ref[...]
    buf = jnp.maximum(bm * sc2_ref[...] + sh2_ref[...], 0.0)
    t = jnp.maximum(_dot(wo1_ref[...], buf) + bo1_ref[...], 0.0)
    validf = (bm[0:1, :] > -1e37).astype(jnp.float32)
    ta = jnp.concatenate([t * validf, validf], axis=0)
    o_ref[...] = _dot(ta.T, waug_ref[...])


def _scale_shift(s, q, cnt, g, b):
    mean = s[:, 0] / cnt
    var = jnp.maximum(q[:, 0] / cnt - mean * mean, 0.0)
    sc = g * jax.lax.rsqrt(var + _EPS)
    sh = b - mean * sc
    return sc[:, None], sh[:, None]


def kernel(polylines, polylines_mask, W_pre, g_pre, b_pre, W1, g1, b1,
           W2, g2, b2, Wo1, bo1, Wo2, bo2):
    B, P, N, C = polylines.shape
    H = W_pre.shape[0]
    O = Wo2.shape[0]
    M = B * P
    f32 = jnp.float32

    xt = polylines.reshape(M, N, C).transpose(1, 2, 0)
    mt = polylines_mask.reshape(M, N).T.reshape(N, 1, M).astype(f32)
    waug = jnp.concatenate([Wo2.T, bo2[None, :]], axis=0)

    grid = (M // _MB,)
    x_spec = pl.BlockSpec((N, C, _MB), lambda i: (0, 0, i))
    m_spec = pl.BlockSpec((N, 1, _MB), lambda i: (0, 0, i))

    def full(shp):
        return pl.BlockSpec(shp, lambda i: tuple(0 for _ in shp))

    accH = pl.BlockSpec((H, 1), lambda i: (0, 0))
    acc1 = pl.BlockSpec((1, 1), lambda i: (0, 0))
    sdH = jax.ShapeDtypeStruct((H, 1), f32)

    s0, q0, c0 = pl.pallas_call(
        _ph1,
        grid=grid,
        in_specs=[x_spec, m_spec, full((H, C))],
        out_specs=[accH, accH, acc1],
        out_shape=[sdH, sdH, jax.ShapeDtypeStruct((1, 1), f32)],
    )(xt, mt, W_pre)

    cnt = jnp.maximum(c0[0, 0], 1.0)
    sc0, sh0 = _scale_shift(s0, q0, cnt, g_pre, b_pre)

    h_spec = pl.BlockSpec((N, H, _MB), lambda i: (0, 0, i))

    s1, q1, h2buf = pl.pallas_call(
        _ph2,
        grid=grid,
        in_specs=[x_spec, m_spec, full((H, C)), full((H, 1)), full((H, 1)),
                  full((H, H)), full((H, H))],
        out_specs=[accH, accH, h_spec],
        out_shape=[sdH, sdH, jax.ShapeDtypeStruct((N, H, M), jnp.bfloat16)],
    )(xt, mt, W_pre, sc0, sh0, W1[:, :H], W1[:, H:])

    sc1, sh1 = _scale_shift(s1, q1, cnt, g1, b1)

    s2, q2, bmax = pl.pallas_call(
        _ph3,
        grid=grid,
        in_specs=[h_spec, m_spec, full((H, 1)), full((H, 1)), full((H, H))],
        out_specs=[accH, accH, pl.BlockSpec((H, _MB), lambda i: (0, i))],
        out_shape=[sdH, sdH, jax.ShapeDtypeStruct((H, M), f32)],
    )(h2buf, mt, sc1, sh1, W2)

    sc2, sh2 = _scale_shift(s2, q2, cnt, g2, b2)

    out = pl.pallas_call(
        _ph4,
        grid=(M // _MB4,),
        in_specs=[pl.BlockSpec((H, _MB4), lambda i: (0, i)),
                  full((H, 1)), full((H, 1)),
                  full((H, H)), full((H, 1)), full((H + 1, O))],
        out_specs=pl.BlockSpec((_MB4, O), lambda i: (i, 0)),
        out_shape=jax.ShapeDtypeStruct((M, O), f32),
    )(bmax, sc2, sh2, Wo1, bo1.reshape(H, 1), waug)

    return out.reshape(B, P, O)


# speedup vs baseline: 1.6851x; 1.0204x over previous
import jax
import jax.numpy as jnp
from jax.experimental import pallas as pl

_EPS = 1e-5
_MB = 1024
_MB4 = 1024
_BIG = 3.0e38


def _acc(ref, val, first):
    @pl.when(first)
    def _():
        ref[...] = jnp.zeros_like(ref)

    ref[...] += val


def _dot(a, b):
    return jnp.dot(a, b, preferred_element_type=jnp.float32)


def _stats(hs, ms, s_ref, q_ref, first):
    hmacc = hs[0] * ms[0]
    qacc = hmacc * hs[0]
    for n in range(1, len(hs)):
        hm = hs[n] * ms[n]
        hmacc = hmacc + hm
        qacc = qacc + hm * hs[n]
    ones = jnp.ones((hmacc.shape[1], 1), jnp.float32)
    _acc(s_ref, _dot(hmacc, ones), first)
    _acc(q_ref, _dot(qacc, ones), first)


def _ph1(x_ref, m_ref, wp_ref, s_ref, q_ref, c_ref):
    first = pl.program_id(0) == 0
    wp = wp_ref[...]
    m = m_ref[...]
    hs = [_dot(wp, x_ref[n]) for n in range(x_ref.shape[0])]
    ms = [m[n] for n in range(m.shape[0])]
    _stats(hs, ms, s_ref, q_ref, first)
    _acc(c_ref, jnp.sum(m).reshape(1, 1), first)


def _ph2(x_ref, m_ref, wp_ref, sc0_ref, sh0_ref, w1a_ref, w1b_ref,
         s_ref, q_ref, h2_ref):
    first = pl.program_id(0) == 0
    wp = wp_ref[...]
    sc0 = sc0_ref[...]
    sh0 = sh0_ref[...]
    m = m_ref[...]
    n_pts = x_ref.shape[0]
    feats = []
    for n in range(n_pts):
        h1 = _dot(wp, x_ref[n])
        feats.append(jnp.maximum(h1 * sc0 + sh0, 0.0) * m[n])
    pooled = feats[0]
    for n in range(1, n_pts):
        pooled = jnp.maximum(pooled, feats[n])
    pb = _dot(w1b_ref[...], pooled)
    w1a = w1a_ref[...]
    h2s = []
    for n in range(n_pts):
        h2 = _dot(w1a, feats[n]) + pb
        h2_ref[n] = h2.astype(jnp.bfloat16)
        h2s.append(h2)
    _stats(h2s, [m[n] for n in range(n_pts)], s_ref, q_ref, first)


def _ph3(h2_ref, m_ref, sc1_ref, sh1_ref, w2_ref, s_ref, q_ref, bm_ref):
    first = pl.program_id(0) == 0
    sc1 = sc1_ref[...]
    sh1 = sh1_ref[...]
    w2 = w2_ref[...]
    m = m_ref[...]
    n_pts = h2_ref.shape[0]
    h3s = []
    bmacc = None
    for n in range(n_pts):
        t2 = jnp.maximum(h2_ref[n].astype(jnp.float32) * sc1 + sh1, 0.0)
        h3 = _dot(w2, t2)
        h3s.append(h3)
        h3m = jnp.where(m[n] != 0.0, h3, -_BIG)
        bmacc = h3m if bmacc is None else jnp.maximum(bmacc, h3m)
    bm_ref[...] = bmacc
    _stats(h3s, [m[n] for n in range(n_pts)], s_ref, q_ref, first)


def _ph4(bm_ref, sc2_ref, sh2_ref, wo1_ref, bo1_ref, waug_ref, o_ref):
    bm = bm_ref[...]
    buf = jnp.maximum(bm * sc2_ref[...] + sh2_ref[...], 0.0)
    t = jnp.maximum(_dot(wo1_ref[...], buf) + bo1_ref[...], 0.0)
    validf = (bm[0:1, :] > -1e37).astype(jnp.float32)
    ta = jnp.concatenate([t * validf, validf], axis=0)
    o_ref[...] = _dot(ta.T, waug_ref[...])


def _scale_shift(s, q, cnt, g, b):
    mean = s[:, 0] / cnt
    var = jnp.maximum(q[:, 0] / cnt - mean * mean, 0.0)
    sc = g * jax.lax.rsqrt(var + _EPS)
    sh = b - mean * sc
    return sc[:, None], sh[:, None]


def kernel(polylines, polylines_mask, W_pre, g_pre, b_pre, W1, g1, b1,
           W2, g2, b2, Wo1, bo1, Wo2, bo2):
    B, P, N, C = polylines.shape
    H = W_pre.shape[0]
    O = Wo2.shape[0]
    M = B * P
    f32 = jnp.float32

    xt = polylines.reshape(M, N, C).transpose(1, 2, 0)
    mt = polylines_mask.reshape(M, N).T.reshape(N, 1, M).astype(f32)
    waug = jnp.concatenate([Wo2.T, bo2[None, :]], axis=0)

    grid = (M // _MB,)
    x_spec = pl.BlockSpec((N, C, _MB), lambda i: (0, 0, i))
    m_spec = pl.BlockSpec((N, 1, _MB), lambda i: (0, 0, i))

    def full(shp):
        return pl.BlockSpec(shp, lambda i: tuple(0 for _ in shp))

    accH = pl.BlockSpec((H, 1), lambda i: (0, 0))
    acc1 = pl.BlockSpec((1, 1), lambda i: (0, 0))
    sdH = jax.ShapeDtypeStruct((H, 1), f32)

    s0, q0, c0 = pl.pallas_call(
        _ph1,
        grid=grid,
        in_specs=[x_spec, m_spec, full((H, C))],
        out_specs=[accH, accH, acc1],
        out_shape=[sdH, sdH, jax.ShapeDtypeStruct((1, 1), f32)],
    )(xt, mt, W_pre)

    cnt = jnp.maximum(c0[0, 0], 1.0)
    sc0, sh0 = _scale_shift(s0, q0, cnt, g_pre, b_pre)

    h_spec = pl.BlockSpec((N, H, _MB), lambda i: (0, 0, i))

    s1, q1, h2buf = pl.pallas_call(
        _ph2,
        grid=grid,
        in_specs=[x_spec, m_spec, full((H, C)), full((H, 1)), full((H, 1)),
                  full((H, H)), full((H, H))],
        out_specs=[accH, accH, h_spec],
        out_shape=[sdH, sdH, jax.ShapeDtypeStruct((N, H, M), jnp.bfloat16)],
    )(xt, mt, W_pre, sc0, sh0, W1[:, :H], W1[:, H:])

    sc1, sh1 = _scale_shift(s1, q1, cnt, g1, b1)

    s2, q2, bmax = pl.pallas_call(
        _ph3,
        grid=grid,
        in_specs=[h_spec, m_spec, full((H, 1)), full((H, 1)), full((H, H))],
        out_specs=[accH, accH, pl.BlockSpec((H, _MB), lambda i: (0, i))],
        out_shape=[sdH, sdH, jax.ShapeDtypeStruct((H, M), f32)],
    )(h2buf, mt, sc1, sh1, W2)

    sc2, sh2 = _scale_shift(s2, q2, cnt, g2, b2)

    out = pl.pallas_call(
        _ph4,
        grid=(M // _MB4,),
        in_specs=[pl.BlockSpec((H, _MB4), lambda i: (0, i)),
                  full((H, 1)), full((H, 1)),
                  full((H, H)), full((H, 1)), full((H + 1, O))],
        out_specs=pl.BlockSpec((_MB4, O), lambda i: (i, 0)),
        out_shape=jax.ShapeDtypeStruct((M, O), f32),
    )(bmax, sc2, sh2, Wo1, bo1.reshape(H, 1), waug)

    return out.reshape(B, P, O)


# speedup vs baseline: 1.7289x; 1.0260x over previous
import jax
import jax.numpy as jnp
from jax.experimental import pallas as pl

_EPS = 1e-5
_MB = 1536
_MB4 = 1024
_BIG = 3.0e38


def _acc(ref, val, first):
    @pl.when(first)
    def _():
        ref[...] = jnp.zeros_like(ref)

    ref[...] += val


def _dot(a, b):
    return jnp.dot(a, b, preferred_element_type=jnp.float32)


def _stats(hs, ms, s_ref, q_ref, first):
    hmacc = hs[0] * ms[0]
    qacc = hmacc * hs[0]
    for n in range(1, len(hs)):
        hm = hs[n] * ms[n]
        hmacc = hmacc + hm
        qacc = qacc + hm * hs[n]
    ones = jnp.ones((hmacc.shape[1], 1), jnp.float32)
    _acc(s_ref, _dot(hmacc, ones), first)
    _acc(q_ref, _dot(qacc, ones), first)


def _ph1(x_ref, m_ref, wp_ref, s_ref, q_ref, c_ref):
    first = pl.program_id(0) == 0
    wp = wp_ref[...]
    m = m_ref[...]
    hs = [_dot(wp, x_ref[n]) for n in range(x_ref.shape[0])]
    ms = [m[n] for n in range(m.shape[0])]
    _stats(hs, ms, s_ref, q_ref, first)
    _acc(c_ref, jnp.sum(m).reshape(1, 1), first)


def _ph2(x_ref, m_ref, wp_ref, sc0_ref, sh0_ref, w1a_ref, w1b_ref,
         s_ref, q_ref, h2_ref):
    first = pl.program_id(0) == 0
    wp = wp_ref[...]
    sc0 = sc0_ref[...]
    sh0 = sh0_ref[...]
    m = m_ref[...]
    n_pts = x_ref.shape[0]
    feats = []
    for n in range(n_pts):
        h1 = _dot(wp, x_ref[n])
        feats.append(jnp.maximum(h1 * sc0 + sh0, 0.0) * m[n])
    pooled = feats[0]
    for n in range(1, n_pts):
        pooled = jnp.maximum(pooled, feats[n])
    pb = _dot(w1b_ref[...], pooled)
    w1a = w1a_ref[...]
    h2s = []
    for n in range(n_pts):
        h2 = _dot(w1a, feats[n]) + pb
        h2_ref[n] = h2.astype(jnp.bfloat16)
        h2s.append(h2)
    _stats(h2s, [m[n] for n in range(n_pts)], s_ref, q_ref, first)


def _ph3(h2_ref, m_ref, sc1_ref, sh1_ref, w2_ref, s_ref, q_ref, bm_ref):
    first = pl.program_id(0) == 0
    sc1 = sc1_ref[...]
    sh1 = sh1_ref[...]
    w2 = w2_ref[...]
    m = m_ref[...]
    n_pts = h2_ref.shape[0]
    h3s = []
    bmacc = None
    for n in range(n_pts):
        t2 = jnp.maximum(h2_ref[n].astype(jnp.float32) * sc1 + sh1, 0.0)
        h3 = _dot(w2, t2)
        h3s.append(h3)
        h3m = jnp.where(m[n] != 0.0, h3, -_BIG)
        bmacc = h3m if bmacc is None else jnp.maximum(bmacc, h3m)
    bm_ref[...] = bmacc
    _stats(h3s, [m[n] for n in range(n_pts)], s_ref, q_ref, first)


def _ph4(bm_ref, sc2_ref, sh2_ref, wo1_ref, bo1_ref, waug_ref, o_ref):
    bm = bm_ref[...]
    buf = jnp.maximum(bm * sc2_ref[...] + sh2_ref[...], 0.0)
    t = jnp.maximum(_dot(wo1_ref[...], buf) + bo1_ref[...], 0.0)
    validf = (bm[0:1, :] > -1e37).astype(jnp.float32)
    ta = jnp.concatenate([t * validf, validf], axis=0)
    o_ref[...] = _dot(ta.T, waug_ref[...])


def _scale_shift(s, q, cnt, g, b):
    mean = s[:, 0] / cnt
    var = jnp.maximum(q[:, 0] / cnt - mean * mean, 0.0)
    sc = g * jax.lax.rsqrt(var + _EPS)
    sh = b - mean * sc
    return sc[:, None], sh[:, None]


def kernel(polylines, polylines_mask, W_pre, g_pre, b_pre, W1, g1, b1,
           W2, g2, b2, Wo1, bo1, Wo2, bo2):
    B, P, N, C = polylines.shape
    H = W_pre.shape[0]
    O = Wo2.shape[0]
    M = B * P
    f32 = jnp.float32

    xt = polylines.reshape(M, N, C).transpose(1, 2, 0)
    mt = polylines_mask.reshape(M, N).T.reshape(N, 1, M).astype(f32)
    waug = jnp.concatenate([Wo2.T, bo2[None, :]], axis=0)

    grid = (M // _MB,)
    x_spec = pl.BlockSpec((N, C, _MB), lambda i: (0, 0, i))
    m_spec = pl.BlockSpec((N, 1, _MB), lambda i: (0, 0, i))

    def full(shp):
        return pl.BlockSpec(shp, lambda i: tuple(0 for _ in shp))

    accH = pl.BlockSpec((H, 1), lambda i: (0, 0))
    acc1 = pl.BlockSpec((1, 1), lambda i: (0, 0))
    sdH = jax.ShapeDtypeStruct((H, 1), f32)

    s0, q0, c0 = pl.pallas_call(
        _ph1,
        grid=grid,
        in_specs=[x_spec, m_spec, full((H, C))],
        out_specs=[accH, accH, acc1],
        out_shape=[sdH, sdH, jax.ShapeDtypeStruct((1, 1), f32)],
    )(xt, mt, W_pre)

    cnt = jnp.maximum(c0[0, 0], 1.0)
    sc0, sh0 = _scale_shift(s0, q0, cnt, g_pre, b_pre)

    h_spec = pl.BlockSpec((N, H, _MB), lambda i: (0, 0, i))

    s1, q1, h2buf = pl.pallas_call(
        _ph2,
        grid=grid,
        in_specs=[x_spec, m_spec, full((H, C)), full((H, 1)), full((H, 1)),
                  full((H, H)), full((H, H))],
        out_specs=[accH, accH, h_spec],
        out_shape=[sdH, sdH, jax.ShapeDtypeStruct((N, H, M), jnp.bfloat16)],
    )(xt, mt, W_pre, sc0, sh0, W1[:, :H], W1[:, H:])

    sc1, sh1 = _scale_shift(s1, q1, cnt, g1, b1)

    s2, q2, bmax = pl.pallas_call(
        _ph3,
        grid=grid,
        in_specs=[h_spec, m_spec, full((H, 1)), full((H, 1)), full((H, H))],
        out_specs=[accH, accH, pl.BlockSpec((H, _MB), lambda i: (0, i))],
        out_shape=[sdH, sdH, jax.ShapeDtypeStruct((H, M), f32)],
    )(h2buf, mt, sc1, sh1, W2)

    sc2, sh2 = _scale_shift(s2, q2, cnt, g2, b2)

    out = pl.pallas_call(
        _ph4,
        grid=(M // _MB4,),
        in_specs=[pl.BlockSpec((H, _MB4), lambda i: (0, i)),
                  full((H, 1)), full((H, 1)),
                  full((H, H)), full((H, 1)), full((H + 1, O))],
        out_specs=pl.BlockSpec((_MB4, O), lambda i: (i, 0)),
        out_shape=jax.ShapeDtypeStruct((M, O), f32),
    )(bmax, sc2, sh2, Wo1, bo1.reshape(H, 1), waug)

    return out.reshape(B, P, O)


# speedup vs baseline: 1.7730x; 1.0255x over previous
import jax
import jax.numpy as jnp
from jax.experimental import pallas as pl

_EPS = 1e-5
_MB = 2048
_MB4 = 2048
_BIG = 3.0e38


def _acc(ref, val, first):
    @pl.when(first)
    def _():
        ref[...] = jnp.zeros_like(ref)

    ref[...] += val


def _dot(a, b):
    return jnp.dot(a, b, preferred_element_type=jnp.float32)


def _stats(hs, ms, s_ref, q_ref, first):
    hmacc = hs[0] * ms[0]
    qacc = hmacc * hs[0]
    for n in range(1, len(hs)):
        hm = hs[n] * ms[n]
        hmacc = hmacc + hm
        qacc = qacc + hm * hs[n]
    ones = jnp.ones((hmacc.shape[1], 1), jnp.float32)
    _acc(s_ref, _dot(hmacc, ones), first)
    _acc(q_ref, _dot(qacc, ones), first)


def _ph1(x_ref, m_ref, wp_ref, s_ref, q_ref, c_ref):
    first = pl.program_id(0) == 0
    wp = wp_ref[...]
    m = m_ref[...]
    hs = [_dot(wp, x_ref[n]) for n in range(x_ref.shape[0])]
    ms = [m[n] for n in range(m.shape[0])]
    _stats(hs, ms, s_ref, q_ref, first)
    _acc(c_ref, jnp.sum(m).reshape(1, 1), first)


def _ph2(x_ref, m_ref, wp_ref, sc0_ref, sh0_ref, w1a_ref, w1b_ref,
         s_ref, q_ref, h2_ref):
    first = pl.program_id(0) == 0
    wp = wp_ref[...]
    sc0 = sc0_ref[...]
    sh0 = sh0_ref[...]
    m = m_ref[...]
    n_pts = x_ref.shape[0]
    feats = []
    for n in range(n_pts):
        h1 = _dot(wp, x_ref[n])
        feats.append(jnp.maximum(h1 * sc0 + sh0, 0.0) * m[n])
    pooled = feats[0]
    for n in range(1, n_pts):
        pooled = jnp.maximum(pooled, feats[n])
    pb = _dot(w1b_ref[...], pooled)
    w1a = w1a_ref[...]
    h2s = []
    for n in range(n_pts):
        h2 = _dot(w1a, feats[n]) + pb
        h2_ref[n] = h2.astype(jnp.bfloat16)
        h2s.append(h2)
    _stats(h2s, [m[n] for n in range(n_pts)], s_ref, q_ref, first)


def _ph3(h2_ref, m_ref, sc1_ref, sh1_ref, w2_ref, s_ref, q_ref, bm_ref):
    first = pl.program_id(0) == 0
    sc1 = sc1_ref[...]
    sh1 = sh1_ref[...]
    w2 = w2_ref[...]
    m = m_ref[...]
    n_pts = h2_ref.shape[0]
    h3s = []
    bmacc = None
    for n in range(n_pts):
        t2 = jnp.maximum(h2_ref[n].astype(jnp.float32) * sc1 + sh1, 0.0)
        h3 = _dot(w2, t2)
        h3s.append(h3)
        h3m = jnp.where(m[n] != 0.0, h3, -_BIG)
        bmacc = h3m if bmacc is None else jnp.maximum(bmacc, h3m)
    bm_ref[...] = bmacc
    _stats(h3s, [m[n] for n in range(n_pts)], s_ref, q_ref, first)


def _ph4(bm_ref, sc2_ref, sh2_ref, wo1_ref, bo1_ref, waug_ref, o_ref):
    bm = bm_ref[...]
    buf = jnp.maximum(bm * sc2_ref[...] + sh2_ref[...], 0.0)
    t = jnp.maximum(_dot(wo1_ref[...], buf) + bo1_ref[...], 0.0)
    validf = (bm[0:1, :] > -1e37).astype(jnp.float32)
    ta = jnp.concatenate([t * validf, validf], axis=0)
    o_ref[...] = _dot(ta.T, waug_ref[...])


def _scale_shift(s, q, cnt, g, b):
    mean = s[:, 0] / cnt
    var = jnp.maximum(q[:, 0] / cnt - mean * mean, 0.0)
    sc = g * jax.lax.rsqrt(var + _EPS)
    sh = b - mean * sc
    return sc[:, None], sh[:, None]


def kernel(polylines, polylines_mask, W_pre, g_pre, b_pre, W1, g1, b1,
           W2, g2, b2, Wo1, bo1, Wo2, bo2):
    B, P, N, C = polylines.shape
    H = W_pre.shape[0]
    O = Wo2.shape[0]
    M = B * P
    f32 = jnp.float32

    xt = polylines.reshape(M, N, C).transpose(1, 2, 0)
    mt = polylines_mask.reshape(M, N).T.reshape(N, 1, M).astype(f32)
    waug = jnp.concatenate([Wo2.T, bo2[None, :]], axis=0)

    grid = (M // _MB,)
    x_spec = pl.BlockSpec((N, C, _MB), lambda i: (0, 0, i))
    m_spec = pl.BlockSpec((N, 1, _MB), lambda i: (0, 0, i))

    def full(shp):
        return pl.BlockSpec(shp, lambda i: tuple(0 for _ in shp))

    accH = pl.BlockSpec((H, 1), lambda i: (0, 0))
    acc1 = pl.BlockSpec((1, 1), lambda i: (0, 0))
    sdH = jax.ShapeDtypeStruct((H, 1), f32)

    s0, q0, c0 = pl.pallas_call(
        _ph1,
        grid=grid,
        in_specs=[x_spec, m_spec, full((H, C))],
        out_specs=[accH, accH, acc1],
        out_shape=[sdH, sdH, jax.ShapeDtypeStruct((1, 1), f32)],
    )(xt, mt, W_pre)

    cnt = jnp.maximum(c0[0, 0], 1.0)
    sc0, sh0 = _scale_shift(s0, q0, cnt, g_pre, b_pre)

    h_spec = pl.BlockSpec((N, H, _MB), lambda i: (0, 0, i))

    s1, q1, h2buf = pl.pallas_call(
        _ph2,
        grid=grid,
        in_specs=[x_spec, m_spec, full((H, C)), full((H, 1)), full((H, 1)),
                  full((H, H)), full((H, H))],
        out_specs=[accH, accH, h_spec],
        out_shape=[sdH, sdH, jax.ShapeDtypeStruct((N, H, M), jnp.bfloat16)],
    )(xt, mt, W_pre, sc0, sh0, W1[:, :H], W1[:, H:])

    sc1, sh1 = _scale_shift(s1, q1, cnt, g1, b1)

    s2, q2, bmax = pl.pallas_call(
        _ph3,
        grid=grid,
        in_specs=[h_spec, m_spec, full((H, 1)), full((H, 1)), full((H, H))],
        out_specs=[accH, accH, pl.BlockSpec((H, _MB), lambda i: (0, i))],
        out_shape=[sdH, sdH, jax.ShapeDtypeStruct((H, M), f32)],
    )(h2buf, mt, sc1, sh1, W2)

    sc2, sh2 = _scale_shift(s2, q2, cnt, g2, b2)

    out = pl.pallas_call(
        _ph4,
        grid=(M // _MB4,),
        in_specs=[pl.BlockSpec((H, _MB4), lambda i: (0, i)),
                  full((H, 1)), full((H, 1)),
                  full((H, H)), full((H, 1)), full((H + 1, O))],
        out_specs=pl.BlockSpec((_MB4, O), lambda i: (i, 0)),
        out_shape=jax.ShapeDtypeStruct((M, O), f32),
    )(bmax, sc2, sh2, Wo1, bo1.reshape(H, 1), waug)

    return out.reshape(B, P, O)


# speedup vs baseline: 1.8094x; 1.0206x over previous
import jax
import jax.numpy as jnp
from jax.experimental import pallas as pl

_EPS = 1e-5
_MB = 3072
_MB4 = 3072
_BIG = 3.0e38


def _acc(ref, val, first):
    @pl.when(first)
    def _():
        ref[...] = jnp.zeros_like(ref)

    ref[...] += val


def _dot(a, b):
    return jnp.dot(a, b, preferred_element_type=jnp.float32)


def _stats(hs, ms, s_ref, q_ref, first):
    hmacc = hs[0] * ms[0]
    qacc = hmacc * hs[0]
    for n in range(1, len(hs)):
        hm = hs[n] * ms[n]
        hmacc = hmacc + hm
        qacc = qacc + hm * hs[n]
    ones = jnp.ones((hmacc.shape[1], 1), jnp.float32)
    _acc(s_ref, _dot(hmacc, ones), first)
    _acc(q_ref, _dot(qacc, ones), first)


def _ph1(x_ref, m_ref, wp_ref, s_ref, q_ref, c_ref):
    first = pl.program_id(0) == 0
    wp = wp_ref[...]
    m = m_ref[...]
    hs = [_dot(wp, x_ref[n]) for n in range(x_ref.shape[0])]
    ms = [m[n] for n in range(m.shape[0])]
    _stats(hs, ms, s_ref, q_ref, first)
    _acc(c_ref, jnp.sum(m).reshape(1, 1), first)


def _ph2(x_ref, m_ref, wp_ref, sc0_ref, sh0_ref, w1a_ref, w1b_ref,
         s_ref, q_ref, h2_ref):
    first = pl.program_id(0) == 0
    wp = wp_ref[...]
    sc0 = sc0_ref[...]
    sh0 = sh0_ref[...]
    m = m_ref[...]
    n_pts = x_ref.shape[0]
    feats = []
    for n in range(n_pts):
        h1 = _dot(wp, x_ref[n])
        feats.append(jnp.maximum(h1 * sc0 + sh0, 0.0) * m[n])
    pooled = feats[0]
    for n in range(1, n_pts):
        pooled = jnp.maximum(pooled, feats[n])
    pb = _dot(w1b_ref[...], pooled)
    w1a = w1a_ref[...]
    h2s = []
    for n in range(n_pts):
        h2 = _dot(w1a, feats[n]) + pb
        h2_ref[n] = h2.astype(jnp.bfloat16)
        h2s.append(h2)
    _stats(h2s, [m[n] for n in range(n_pts)], s_ref, q_ref, first)


def _ph3(h2_ref, m_ref, sc1_ref, sh1_ref, w2_ref, s_ref, q_ref, bm_ref):
    first = pl.program_id(0) == 0
    sc1 = sc1_ref[...]
    sh1 = sh1_ref[...]
    w2 = w2_ref[...]
    m = m_ref[...]
    n_pts = h2_ref.shape[0]
    h3s = []
    bmacc = None
    for n in range(n_pts):
        t2 = jnp.maximum(h2_ref[n].astype(jnp.float32) * sc1 + sh1, 0.0)
        h3 = _dot(w2, t2)
        h3s.append(h3)
        h3m = jnp.where(m[n] != 0.0, h3, -_BIG)
        bmacc = h3m if bmacc is None else jnp.maximum(bmacc, h3m)
    bm_ref[...] = bmacc
    _stats(h3s, [m[n] for n in range(n_pts)], s_ref, q_ref, first)


def _ph4(bm_ref, sc2_ref, sh2_ref, wo1_ref, bo1_ref, waug_ref, o_ref):
    bm = bm_ref[...]
    buf = jnp.maximum(bm * sc2_ref[...] + sh2_ref[...], 0.0)
    t = jnp.maximum(_dot(wo1_ref[...], buf) + bo1_ref[...], 0.0)
    validf = (bm[0:1, :] > -1e37).astype(jnp.float32)
    ta = jnp.concatenate([t * validf, validf], axis=0)
    o_ref[...] = _dot(ta.T, waug_ref[...])


def _scale_shift(s, q, cnt, g, b):
    mean = s[:, 0] / cnt
    var = jnp.maximum(q[:, 0] / cnt - mean * mean, 0.0)
    sc = g * jax.lax.rsqrt(var + _EPS)
    sh = b - mean * sc
    return sc[:, None], sh[:, None]


def kernel(polylines, polylines_mask, W_pre, g_pre, b_pre, W1, g1, b1,
           W2, g2, b2, Wo1, bo1, Wo2, bo2):
    B, P, N, C = polylines.shape
    H = W_pre.shape[0]
    O = Wo2.shape[0]
    M = B * P
    f32 = jnp.float32

    xt = polylines.reshape(M, N, C).transpose(1, 2, 0)
    mt = polylines_mask.reshape(M, N).T.reshape(N, 1, M).astype(f32)
    waug = jnp.concatenate([Wo2.T, bo2[None, :]], axis=0)

    grid = (M // _MB,)
    x_spec = pl.BlockSpec((N, C, _MB), lambda i: (0, 0, i))
    m_spec = pl.BlockSpec((N, 1, _MB), lambda i: (0, 0, i))

    def full(shp):
        return pl.BlockSpec(shp, lambda i: tuple(0 for _ in shp))

    accH = pl.BlockSpec((H, 1), lambda i: (0, 0))
    acc1 = pl.BlockSpec((1, 1), lambda i: (0, 0))
    sdH = jax.ShapeDtypeStruct((H, 1), f32)

    s0, q0, c0 = pl.pallas_call(
        _ph1,
        grid=grid,
        in_specs=[x_spec, m_spec, full((H, C))],
        out_specs=[accH, accH, acc1],
        out_shape=[sdH, sdH, jax.ShapeDtypeStruct((1, 1), f32)],
    )(xt, mt, W_pre)

    cnt = jnp.maximum(c0[0, 0], 1.0)
    sc0, sh0 = _scale_shift(s0, q0, cnt, g_pre, b_pre)

    h_spec = pl.BlockSpec((N, H, _MB), lambda i: (0, 0, i))

    s1, q1, h2buf = pl.pallas_call(
        _ph2,
        grid=grid,
        in_specs=[x_spec, m_spec, full((H, C)), full((H, 1)), full((H, 1)),
                  full((H, H)), full((H, H))],
        out_specs=[accH, accH, h_spec],
        out_shape=[sdH, sdH, jax.ShapeDtypeStruct((N, H, M), jnp.bfloat16)],
    )(xt, mt, W_pre, sc0, sh0, W1[:, :H], W1[:, H:])

    sc1, sh1 = _scale_shift(s1, q1, cnt, g1, b1)

    s2, q2, bmax = pl.pallas_call(
        _ph3,
        grid=grid,
        in_specs=[h_spec, m_spec, full((H, 1)), full((H, 1)), full((H, H))],
        out_specs=[accH, accH, pl.BlockSpec((H, _MB), lambda i: (0, i))],
        out_shape=[sdH, sdH, jax.ShapeDtypeStruct((H, M), f32)],
    )(h2buf, mt, sc1, sh1, W2)

    sc2, sh2 = _scale_shift(s2, q2, cnt, g2, b2)

    out = pl.pallas_call(
        _ph4,
        grid=(M // _MB4,),
        in_specs=[pl.BlockSpec((H, _MB4), lambda i: (0, i)),
                  full((H, 1)), full((H, 1)),
                  full((H, H)), full((H, 1)), full((H + 1, O))],
        out_specs=pl.BlockSpec((_MB4, O), lambda i: (i, 0)),
        out_shape=jax.ShapeDtypeStruct((M, O), f32),
    )(bmax, sc2, sh2, Wo1, bo1.reshape(H, 1), waug)

    return out.reshape(B, P, O)
